# Initial kernel scaffold; baseline (speedup 1.0000x reference)
import jax
import jax.numpy as jnp
from jax.experimental import pallas as pl


def kernel(tokens, t, emb):
    raise NotImplementedError("write your pallas kernel here")


# speedup vs baseline: 1.0106x; 1.0106x over previous
import jax
import jax.numpy as jnp
from jax.experimental import pallas as pl
from jax.experimental.pallas import tpu as pltpu


def _add_row_kernel(t_ref, x_ref, emb_ref, o_ref):
    idx = t_ref[0]
    row = emb_ref[pl.ds(idx, 1), :]
    o_ref[...] = x_ref[...] + row


def kernel(tokens, t, emb):
    B, S, D = tokens.shape
    flat = tokens.reshape(B * S, D)
    R = B * S
    BLK = 1024
    grid = (R // BLK,)
    t_arr = jnp.asarray(t, dtype=jnp.int32).reshape(1)
    out = pl.pallas_call(
        _add_row_kernel,
        grid=grid,
        in_specs=[
            pl.BlockSpec(memory_space=pltpu.SMEM),
            pl.BlockSpec((BLK, D), lambda i: (i, 0)),
            pl.BlockSpec((emb.shape[0], D), lambda i: (0, 0)),
        ],
        out_specs=pl.BlockSpec((BLK, D), lambda i: (i, 0)),
        out_shape=jax.ShapeDtypeStruct((R, D), tokens.dtype),
    )(t_arr, flat, emb)
    return out.reshape(B, S, D)
